# baseline (speedup 1.0000x reference)
import functools

import jax
import jax.numpy as jnp
from jax import lax
from jax.experimental import pallas as pl
from jax.experimental.pallas import tpu as pltpu, tpu_sc as plsc

N = 10000
D = 128
H = 128
C = 40
E = 320000
B = 1000

NC = 2
NS = 16
NPAD = 10240
ROWS_PER_TILE = NPAD // NS
TRASH = N
CHUNK = 128
CHUNKS_PER_TILE = 160
NCHUNKS = NS * CHUNKS_PER_TILE
EP = NCHUNKS * CHUNK
QP = 1024
Q_PER_TILE = QP // NS
DH = D // NC
GW = 64
RING = 4
PADC = (EP - E) // CHUNK

_mesh = plsc.VectorSubcoreMesh(core_axis_name="c", subcore_axis_name="s",
                               num_cores=NC, num_subcores=NS)


def _zero_2d(ref, rows, cols):
    def zrow(i, _):
        for c in range(cols // 16):
            ref[i, pl.ds(c * 16, 16)] = jnp.zeros((16,), jnp.float32)
        return 0
    lax.fori_loop(0, rows, zrow, 0)


def _ring_pipeline(table, sidx, didx, acc, nchunks, bufs, gsems, ssems,
                   extra=None):
    for k in range(2):
        pltpu.async_copy(table.at[sidx.at[k]], bufs[k], gsems[k])

    def macro(m, _):
        for k in range(RING):
            j = RING * m + k
            pltpu.make_async_copy(table.at[sidx.at[j]], bufs[k],
                                  gsems[k]).wait()
            pltpu.async_copy(bufs[k], acc.at[didx.at[j]], ssems[k],
                             add=True)
            if extra is not None:
                extra(k, j)
            bn = (k + 2) % RING
            jn = j + 2

            @pl.when(jn < nchunks)
            def _():
                @pl.when(j >= 2)
                def _():
                    pltpu.make_async_copy(bufs[bn], acc.at[didx.at[0]],
                                          ssems[bn]).wait()

                pltpu.async_copy(table.at[sidx.at[jn]], bufs[bn], gsems[bn])

        return 0

    lax.fori_loop(0, nchunks // RING, macro, 0)
    for k in (2, 3):
        pltpu.make_async_copy(bufs[k], acc.at[didx.at[0]], ssems[k]).wait()


def _seg_sum_kernel(emb2_hbm, edges_hbm, sum_out, cnt_out,
                    sidx, didx, b0, b1, b2, b3, ones, zc, tmp,
                    acc, cacc, g0, g1, g2, g3, s0, s1, s2, s3, semC):
    c_id = lax.axis_index("c")
    t = lax.axis_index("s")
    base = t * ROWS_PER_TILE

    _zero_2d(tmp, 64, DH)
    for c in range(CHUNK // 16):
        ones[pl.ds(c * 16, 16)] = jnp.ones((16,), jnp.float32)
    for c in range(ROWS_PER_TILE // 16):
        zc[pl.ds(c * 16, 16)] = jnp.zeros((16,), jnp.float32)

    for kk in range(ROWS_PER_TILE // 64):
        pltpu.sync_copy(tmp, acc.at[pl.ds(base + kk * 64, 64)])
    pltpu.sync_copy(zc, cacc.at[pl.ds(base, ROWS_PER_TILE)])

    pltpu.sync_copy(
        edges_hbm.at[0, pl.ds(t * CHUNKS_PER_TILE, CHUNKS_PER_TILE)], sidx)
    pltpu.sync_copy(
        edges_hbm.at[1, pl.ds(t * CHUNKS_PER_TILE, CHUNKS_PER_TILE)], didx)
    off = jnp.full((16,), c_id * (N + 8), jnp.int32)

    def arow(i, _):
        for c in range(CHUNK // 16):
            sl = pl.ds(c * 16, 16)
            sidx[i, sl] = sidx[i, sl] + off
        return 0

    lax.fori_loop(0, CHUNKS_PER_TILE, arow, 0)
    plsc.subcore_barrier()

    is_last = t == NS - 1

    def cnt_extra(k, j):
        @pl.when((c_id == (k % 2)) & jnp.logical_not(
            is_last & (j >= CHUNKS_PER_TILE - PADC)))
        def _():
            pltpu.async_copy(ones, cacc.at[didx.at[j]], semC, add=True)

    _ring_pipeline(emb2_hbm, sidx, didx, acc, CHUNKS_PER_TILE,
                   (b0, b1, b2, b3), (g0, g1, g2, g3), (s0, s1, s2, s3),
                   extra=cnt_extra)

    def drainC(i, _):
        pltpu.make_async_copy(ones, cacc.at[didx.at[0]], semC).wait()
        return 0

    n_drain = jnp.where(is_last, (CHUNKS_PER_TILE - PADC) // 2,
                        CHUNKS_PER_TILE // 2)
    lax.fori_loop(0, n_drain, drainC, 0)
    plsc.subcore_barrier()

    obase = c_id * NPAD + base
    pltpu.sync_copy(acc.at[pl.ds(base, ROWS_PER_TILE)],
                    sum_out.at[pl.ds(obase, ROWS_PER_TILE)])
    pltpu.sync_copy(cacc.at[pl.ds(base, ROWS_PER_TILE)],
                    cnt_out.at[pl.ds(obase, ROWS_PER_TILE)])


_seg_sum = functools.partial(
    pl.kernel, _seg_sum_kernel, mesh=_mesh,
    compiler_params=pltpu.CompilerParams(use_tc_tiling_on_sc=False),
    out_type=[jax.ShapeDtypeStruct((NC * NPAD, DH), jnp.float32),
              jax.ShapeDtypeStruct((NC * NPAD,), jnp.float32)],
    scratch_types=[
        pltpu.VMEM((CHUNKS_PER_TILE, CHUNK), jnp.int32),
        pltpu.VMEM((CHUNKS_PER_TILE, CHUNK), jnp.int32),
        pltpu.VMEM((CHUNK, DH), jnp.float32),
        pltpu.VMEM((CHUNK, DH), jnp.float32),
        pltpu.VMEM((CHUNK, DH), jnp.float32),
        pltpu.VMEM((CHUNK, DH), jnp.float32),
        pltpu.VMEM((CHUNK,), jnp.float32),
        pltpu.VMEM((ROWS_PER_TILE,), jnp.float32),
        pltpu.VMEM((64, DH), jnp.float32),
        pltpu.VMEM_SHARED((NPAD, DH), jnp.float32),
        pltpu.VMEM_SHARED((NPAD,), jnp.float32),
        pltpu.SemaphoreType.DMA,
        pltpu.SemaphoreType.DMA,
        pltpu.SemaphoreType.DMA,
        pltpu.SemaphoreType.DMA,
        pltpu.SemaphoreType.DMA,
        pltpu.SemaphoreType.DMA,
        pltpu.SemaphoreType.DMA,
        pltpu.SemaphoreType.DMA,
        pltpu.SemaphoreType.DMA,
    ])()


def _dense_kernel(x_ref, sa_ref, sb_ref, ca_ref, cb_ref, wl1a_ref, wl1b_ref,
                  wr1_ref, b1_ref, wl2_ref, wr2_ref, b2_ref,
                  g_ref, hr_ref, ivb_ref):
    dn = (((1,), (1,)), ((), ()))
    mm = lambda a, b: lax.dot_general(
        a, b, dn, preferred_element_type=jnp.float32)
    cnt = ca_ref[...] + cb_ref[...]
    iv = 1.0 / jnp.maximum(cnt, 1.0)
    h = mm(sa_ref[...] * iv, wl1a_ref[...])
    h = h + mm(sb_ref[...] * iv, wl1b_ref[...])
    h = h + mm(x_ref[...], wr1_ref[...])
    h = jnp.maximum(h + b1_ref[...], 0.0)
    rows = lax.broadcasted_iota(jnp.int32, (h.shape[0], 1), 0) \
        + pl.program_id(0) * h.shape[0]
    g_ref[...] = jnp.where(rows < N, mm(h, wl2_ref[...]), 0.0)
    hr_ref[...] = mm(h, wr2_ref[...]) + b2_ref[...]
    ivb_ref[...] = jnp.broadcast_to(iv, ivb_ref.shape)


def _dense(x, s2, c2, wl1a, wl1b, wr1, b1, wl2, wr2, b2):
    RB = 1280
    nb = NPAD // RB
    grid = (nb,)
    full = lambda shape: pl.BlockSpec(shape, lambda i: tuple(0 for _ in shape))
    row = lambda cols: pl.BlockSpec((RB, cols), lambda i: (i, 0))
    hi = lambda cols: pl.BlockSpec((RB, cols), lambda i: (nb + i, 0))
    return pl.pallas_call(
        _dense_kernel,
        grid=grid,
        in_specs=[
            row(D), row(DH), hi(DH), row(1), hi(1),
            full((H, DH)), full((H, DH)), full((H, D)), full((1, H)),
            full((GW, H)), full((GW, H)), full((1, GW)),
        ],
        out_specs=[row(GW), row(GW), row(16)],
        out_shape=[jax.ShapeDtypeStruct((NPAD, GW), jnp.float32),
                   jax.ShapeDtypeStruct((NPAD, GW), jnp.float32),
                   jax.ShapeDtypeStruct((NPAD, 16), jnp.float32)],
    )(x, s2, s2, c2, c2, wl1a, wl1b, wr1, b1, wl2, wr2, b2)


def _layer2_kernel(g_hbm, edges_hbm, hr_hbm, ivb_hbm, nodes_hbm, out_hbm,
                   sidx, didx, b0, b1, b2, b3, tmp, nbuf, qsum, qhr, qiv,
                   qout, acc, g0, g1, g2, g3, s0, s1, s2, s3):
    c_id = lax.axis_index("c")
    t = lax.axis_index("s")
    base = t * ROWS_PER_TILE
    half = CHUNKS_PER_TILE // NC
    cbase = t * CHUNKS_PER_TILE + c_id * half

    _zero_2d(tmp, 64, GW)
    for kk in range(ROWS_PER_TILE // 64):
        pltpu.sync_copy(tmp, acc.at[pl.ds(base + kk * 64, 64)])

    pltpu.sync_copy(edges_hbm.at[0, pl.ds(cbase, half)], sidx)
    pltpu.sync_copy(edges_hbm.at[1, pl.ds(cbase, half)], didx)
    plsc.subcore_barrier()

    _ring_pipeline(g_hbm, sidx, didx, acc, half, (b0, b1, b2, b3),
                   (g0, g1, g2, g3), (s0, s1, s2, s3))
    plsc.subcore_barrier()

    pltpu.sync_copy(nodes_hbm.at[pl.ds(t * Q_PER_TILE, Q_PER_TILE)], nbuf)
    _zero_2d(qhr, Q_PER_TILE, GW)

    @pl.when(c_id == 0)
    def _():
        pltpu.async_copy(hr_hbm.at[nbuf], qhr, g0).wait()

    pltpu.async_copy(acc.at[nbuf], qsum, g0).wait()
    pltpu.async_copy(ivb_hbm.at[nbuf], qiv, g0).wait()

    def qrow(q, _):
        iv = qiv[q, pl.ds(0, 16)]
        for c in range(GW // 16):
            sl = pl.ds(c * 16, 16)
            qout[q, sl] = qsum[q, sl] * iv + qhr[q, sl]
        return 0

    lax.fori_loop(0, Q_PER_TILE, qrow, 0)
    pltpu.sync_copy(qout, out_hbm.at[c_id, pl.ds(t * Q_PER_TILE, Q_PER_TILE)])


_layer2 = functools.partial(
    pl.kernel, _layer2_kernel, mesh=_mesh,
    compiler_params=pltpu.CompilerParams(use_tc_tiling_on_sc=False),
    out_type=jax.ShapeDtypeStruct((NC, QP, GW), jnp.float32),
    scratch_types=[
        pltpu.VMEM((CHUNKS_PER_TILE // NC, CHUNK), jnp.int32),
        pltpu.VMEM((CHUNKS_PER_TILE // NC, CHUNK), jnp.int32),
        pltpu.VMEM((CHUNK, GW), jnp.float32),
        pltpu.VMEM((CHUNK, GW), jnp.float32),
        pltpu.VMEM((CHUNK, GW), jnp.float32),
        pltpu.VMEM((CHUNK, GW), jnp.float32),
        pltpu.VMEM((64, GW), jnp.float32),
        pltpu.VMEM((Q_PER_TILE,), jnp.int32),
        pltpu.VMEM((Q_PER_TILE, GW), jnp.float32),
        pltpu.VMEM((Q_PER_TILE, GW), jnp.float32),
        pltpu.VMEM((Q_PER_TILE, 16), jnp.float32),
        pltpu.VMEM((Q_PER_TILE, GW), jnp.float32),
        pltpu.VMEM_SHARED((NPAD, GW), jnp.float32),
        pltpu.SemaphoreType.DMA,
        pltpu.SemaphoreType.DMA,
        pltpu.SemaphoreType.DMA,
        pltpu.SemaphoreType.DMA,
        pltpu.SemaphoreType.DMA,
        pltpu.SemaphoreType.DMA,
        pltpu.SemaphoreType.DMA,
        pltpu.SemaphoreType.DMA,
    ])()


def kernel(emb, W_l1, b1, W_r1, W_l2, b2, W_r2, nodes, edge_index):
    pad = EP - E
    padcols = jnp.concatenate(
        [jnp.full((1, pad), N, jnp.int32),
         ((jnp.arange(pad, dtype=jnp.int32) * 131) % NPAD).reshape(1, pad)])
    edges3 = jnp.concatenate(
        [edge_index.astype(jnp.int32), padcols], axis=1).reshape(
            2, NCHUNKS, CHUNK)
    nodesp = jnp.concatenate(
        [nodes.astype(jnp.int32), jnp.zeros((QP - B,), jnp.int32)])

    zrows = jnp.zeros((8, DH), jnp.float32)
    emb2 = jnp.concatenate([emb[:, :DH], zrows, emb[:, DH:], zrows], axis=0)
    sum2, cnt2 = _seg_sum(emb2, edges3)

    embp = jnp.pad(emb, ((0, NPAD - N), (0, 0)))
    wl2p = jnp.pad(W_l2, ((0, GW - C), (0, 0)))
    wr2p = jnp.pad(W_r2, ((0, GW - C), (0, 0)))
    b2p = jnp.pad(b2, (0, GW - C)).reshape(1, GW)
    g, hr, ivb = _dense(
        embp, sum2, cnt2.reshape(NC * NPAD, 1),
        W_l1[:, :DH], W_l1[:, DH:], W_r1, b1.reshape(1, H), wl2p, wr2p, b2p)

    outq = _layer2(g, edges3, hr, ivb, nodesp)
    return (outq[0] + outq[1])[:B, :C]


# speedup vs baseline: 2.4800x; 2.4800x over previous
import functools

import jax
import jax.numpy as jnp
from jax import lax
from jax.experimental import pallas as pl
from jax.experimental.pallas import tpu as pltpu, tpu_sc as plsc

N = 10000
D = 128
H = 128
C = 40
E = 320000
B = 1000

NC = 2
NS = 16
NPAD = 10240
ROWS_PER_TILE = NPAD // NS
TRASH = N
CHUNK = 128
CHUNKS_PER_TILE = 160
NCHUNKS = NS * CHUNKS_PER_TILE
EP = NCHUNKS * CHUNK
QP = 1024
Q_PER_TILE = QP // NS
DH = D // NC
GW = 64
RING = 4
PADC = (EP - E) // CHUNK

_mesh = plsc.VectorSubcoreMesh(core_axis_name="c", subcore_axis_name="s",
                               num_cores=NC, num_subcores=NS)


def _zero_2d(ref, rows, cols):
    def zrow(i, _):
        for c in range(cols // 16):
            ref[i, pl.ds(c * 16, 16)] = jnp.zeros((16,), jnp.float32)
        return 0
    lax.fori_loop(0, rows, zrow, 0)


def _ring_pipeline(table, sidx, didx, acc, nchunks, bufs, gsems, ssems,
                   extra=None):
    for k in range(2):
        pltpu.async_copy(table.at[sidx.at[k]], bufs[k], gsems[k])

    def macro(m, _):
        for k in range(RING):
            j = RING * m + k
            pltpu.make_async_copy(table.at[sidx.at[j]], bufs[k],
                                  gsems[k]).wait()
            pltpu.async_copy(bufs[k], acc.at[didx.at[j]], ssems[k],
                             add=True)
            if extra is not None:
                extra(k, j)
            bn = (k + 2) % RING
            jn = j + 2

            @pl.when(jn < nchunks)
            def _():
                @pl.when(j >= 2)
                def _():
                    pltpu.make_async_copy(bufs[bn], acc.at[didx.at[0]],
                                          ssems[bn]).wait()

                pltpu.async_copy(table.at[sidx.at[jn]], bufs[bn], gsems[bn])

        return 0

    lax.fori_loop(0, nchunks // RING, macro, 0)
    for k in (2, 3):
        pltpu.make_async_copy(bufs[k], acc.at[didx.at[0]], ssems[k]).wait()


def _seg_sum_kernel(emb2_hbm, edges_hbm, sum_out, cnt_out,
                    sidx, didx, b0, b1, b2, b3, ones, zc, tmp,
                    acc, cacc, g0, g1, g2, g3, s0, s1, s2, s3, semC):
    c_id = lax.axis_index("c")
    t = lax.axis_index("s")
    base = t * ROWS_PER_TILE

    _zero_2d(tmp, 64, DH)
    for c in range(CHUNK // 16):
        ones[pl.ds(c * 16, 16)] = jnp.ones((16,), jnp.float32)
    for c in range(ROWS_PER_TILE // 16):
        zc[pl.ds(c * 16, 16)] = jnp.zeros((16,), jnp.float32)

    for kk in range(ROWS_PER_TILE // 64):
        pltpu.sync_copy(tmp, acc.at[pl.ds(base + kk * 64, 64)])
    pltpu.sync_copy(zc, cacc.at[pl.ds(base, ROWS_PER_TILE)])

    pltpu.sync_copy(
        edges_hbm.at[0, pl.ds(t * CHUNKS_PER_TILE, CHUNKS_PER_TILE)], sidx)
    pltpu.sync_copy(
        edges_hbm.at[1, pl.ds(t * CHUNKS_PER_TILE, CHUNKS_PER_TILE)], didx)
    off = jnp.full((16,), c_id * N, jnp.int32)

    def arow(i, _):
        for c in range(CHUNK // 16):
            sl = pl.ds(c * 16, 16)
            sidx[i, sl] = sidx[i, sl] + off
        return 0

    lax.fori_loop(0, CHUNKS_PER_TILE, arow, 0)
    plsc.subcore_barrier()

    def cnt_extra(k, j):
        @pl.when(c_id == (k % 2))
        def _():
            pltpu.async_copy(ones, cacc.at[didx.at[j]], semC, add=True)

    _ring_pipeline(emb2_hbm, sidx, didx, acc, CHUNKS_PER_TILE,
                   (b0, b1, b2, b3), (g0, g1, g2, g3), (s0, s1, s2, s3),
                   extra=cnt_extra)

    def drainC(i, _):
        pltpu.make_async_copy(ones, cacc.at[didx.at[0]], semC).wait()
        return 0

    lax.fori_loop(0, CHUNKS_PER_TILE // 2, drainC, 0)
    plsc.subcore_barrier()

    obase = c_id * NPAD + base
    pltpu.sync_copy(acc.at[pl.ds(base, ROWS_PER_TILE)],
                    sum_out.at[pl.ds(obase, ROWS_PER_TILE)])
    pltpu.sync_copy(cacc.at[pl.ds(base, ROWS_PER_TILE)],
                    cnt_out.at[pl.ds(obase, ROWS_PER_TILE)])


_seg_sum = functools.partial(
    pl.kernel, _seg_sum_kernel, mesh=_mesh,
    compiler_params=pltpu.CompilerParams(use_tc_tiling_on_sc=False),
    out_type=[jax.ShapeDtypeStruct((NC * NPAD, DH), jnp.float32),
              jax.ShapeDtypeStruct((NC * NPAD,), jnp.float32)],
    scratch_types=[
        pltpu.VMEM((CHUNKS_PER_TILE, CHUNK), jnp.int32),
        pltpu.VMEM((CHUNKS_PER_TILE, CHUNK), jnp.int32),
        pltpu.VMEM((CHUNK, DH), jnp.float32),
        pltpu.VMEM((CHUNK, DH), jnp.float32),
        pltpu.VMEM((CHUNK, DH), jnp.float32),
        pltpu.VMEM((CHUNK, DH), jnp.float32),
        pltpu.VMEM((CHUNK,), jnp.float32),
        pltpu.VMEM((ROWS_PER_TILE,), jnp.float32),
        pltpu.VMEM((64, DH), jnp.float32),
        pltpu.VMEM_SHARED((NPAD, DH), jnp.float32),
        pltpu.VMEM_SHARED((NPAD,), jnp.float32),
        pltpu.SemaphoreType.DMA,
        pltpu.SemaphoreType.DMA,
        pltpu.SemaphoreType.DMA,
        pltpu.SemaphoreType.DMA,
        pltpu.SemaphoreType.DMA,
        pltpu.SemaphoreType.DMA,
        pltpu.SemaphoreType.DMA,
        pltpu.SemaphoreType.DMA,
        pltpu.SemaphoreType.DMA,
    ])()


def _dense_kernel(x_ref, sa_ref, sb_ref, ca_ref, cb_ref, wl1a_ref, wl1b_ref,
                  wr1_ref, b1_ref, wl2_ref, wr2_ref, b2_ref,
                  g_ref, hr_ref, ivb_ref):
    dn = (((1,), (1,)), ((), ()))
    mm = lambda a, b: lax.dot_general(
        a, b, dn, preferred_element_type=jnp.float32)
    cnt = ca_ref[...] + cb_ref[...]
    iv = 1.0 / jnp.maximum(cnt, 1.0)
    h = mm(sa_ref[...] * iv, wl1a_ref[...])
    h = h + mm(sb_ref[...] * iv, wl1b_ref[...])
    h = h + mm(x_ref[...], wr1_ref[...])
    h = jnp.maximum(h + b1_ref[...], 0.0)
    g_ref[...] = mm(h, wl2_ref[...])
    hr_ref[...] = mm(h, wr2_ref[...]) + b2_ref[...]
    ivb_ref[...] = jnp.broadcast_to(iv, ivb_ref.shape)


def _dense(x, s2, c2, wl1a, wl1b, wr1, b1, wl2, wr2, b2):
    RB = 1280
    nb = NPAD // RB
    grid = (nb,)
    full = lambda shape: pl.BlockSpec(shape, lambda i: tuple(0 for _ in shape))
    row = lambda cols: pl.BlockSpec((RB, cols), lambda i: (i, 0))
    hi = lambda cols: pl.BlockSpec((RB, cols), lambda i: (nb + i, 0))
    return pl.pallas_call(
        _dense_kernel,
        grid=grid,
        in_specs=[
            row(D), row(DH), hi(DH), row(1), hi(1),
            full((H, DH)), full((H, DH)), full((H, D)), full((1, H)),
            full((GW, H)), full((GW, H)), full((1, GW)),
        ],
        out_specs=[row(GW), row(GW), row(16)],
        out_shape=[jax.ShapeDtypeStruct((NPAD, GW), jnp.float32),
                   jax.ShapeDtypeStruct((NPAD, GW), jnp.float32),
                   jax.ShapeDtypeStruct((NPAD, 16), jnp.float32)],
    )(x, s2, s2, c2, c2, wl1a, wl1b, wr1, b1, wl2, wr2, b2)


def _layer2_kernel(g_hbm, edges_hbm, hr_hbm, ivb_hbm, nodes_hbm, out_hbm,
                   sidx, didx, b0, b1, b2, b3, tmp, nbuf, qsum, qhr, qiv,
                   qout, acc, g0, g1, g2, g3, s0, s1, s2, s3):
    c_id = lax.axis_index("c")
    t = lax.axis_index("s")
    base = t * ROWS_PER_TILE
    half = CHUNKS_PER_TILE // NC
    cbase = t * CHUNKS_PER_TILE + c_id * half

    _zero_2d(tmp, 64, GW)
    for kk in range(ROWS_PER_TILE // 64):
        pltpu.sync_copy(tmp, acc.at[pl.ds(base + kk * 64, 64)])

    pltpu.sync_copy(edges_hbm.at[0, pl.ds(cbase, half)], sidx)
    pltpu.sync_copy(edges_hbm.at[1, pl.ds(cbase, half)], didx)
    plsc.subcore_barrier()

    _ring_pipeline(g_hbm, sidx, didx, acc, half, (b0, b1, b2, b3),
                   (g0, g1, g2, g3), (s0, s1, s2, s3))
    plsc.subcore_barrier()

    pltpu.sync_copy(nodes_hbm.at[pl.ds(t * Q_PER_TILE, Q_PER_TILE)], nbuf)
    _zero_2d(qhr, Q_PER_TILE, GW)

    @pl.when(c_id == 0)
    def _():
        pltpu.async_copy(hr_hbm.at[nbuf], qhr, g0).wait()

    pltpu.async_copy(acc.at[nbuf], qsum, g0).wait()
    pltpu.async_copy(ivb_hbm.at[nbuf], qiv, g0).wait()

    def qrow(q, _):
        iv = qiv[q, pl.ds(0, 16)]
        for c in range(GW // 16):
            sl = pl.ds(c * 16, 16)
            qout[q, sl] = qsum[q, sl] * iv + qhr[q, sl]
        return 0

    lax.fori_loop(0, Q_PER_TILE, qrow, 0)
    pltpu.sync_copy(qout, out_hbm.at[c_id, pl.ds(t * Q_PER_TILE, Q_PER_TILE)])


_layer2 = functools.partial(
    pl.kernel, _layer2_kernel, mesh=_mesh,
    compiler_params=pltpu.CompilerParams(use_tc_tiling_on_sc=False),
    out_type=jax.ShapeDtypeStruct((NC, QP, GW), jnp.float32),
    scratch_types=[
        pltpu.VMEM((CHUNKS_PER_TILE // NC, CHUNK), jnp.int32),
        pltpu.VMEM((CHUNKS_PER_TILE // NC, CHUNK), jnp.int32),
        pltpu.VMEM((CHUNK, GW), jnp.float32),
        pltpu.VMEM((CHUNK, GW), jnp.float32),
        pltpu.VMEM((CHUNK, GW), jnp.float32),
        pltpu.VMEM((CHUNK, GW), jnp.float32),
        pltpu.VMEM((64, GW), jnp.float32),
        pltpu.VMEM((Q_PER_TILE,), jnp.int32),
        pltpu.VMEM((Q_PER_TILE, GW), jnp.float32),
        pltpu.VMEM((Q_PER_TILE, GW), jnp.float32),
        pltpu.VMEM((Q_PER_TILE, 16), jnp.float32),
        pltpu.VMEM((Q_PER_TILE, GW), jnp.float32),
        pltpu.VMEM_SHARED((NPAD, GW), jnp.float32),
        pltpu.SemaphoreType.DMA,
        pltpu.SemaphoreType.DMA,
        pltpu.SemaphoreType.DMA,
        pltpu.SemaphoreType.DMA,
        pltpu.SemaphoreType.DMA,
        pltpu.SemaphoreType.DMA,
        pltpu.SemaphoreType.DMA,
        pltpu.SemaphoreType.DMA,
    ])()


def kernel(emb, W_l1, b1, W_r1, W_l2, b2, W_r2, nodes, edge_index):
    pad = EP - E
    ar = jnp.arange(pad, dtype=jnp.int32)
    padcols = jnp.concatenate(
        [((ar * 9973) % N).reshape(1, pad),
         (N + (ar * 131) % (NPAD - N)).reshape(1, pad)])
    edges3 = jnp.concatenate(
        [edge_index.astype(jnp.int32), padcols], axis=1).reshape(
            2, NCHUNKS, CHUNK)
    nodesp = jnp.concatenate(
        [nodes.astype(jnp.int32), jnp.zeros((QP - B,), jnp.int32)])

    emb2 = jnp.concatenate([emb[:, :DH], emb[:, DH:]], axis=0)
    sum2, cnt2 = _seg_sum(emb2, edges3)

    embp = jnp.pad(emb, ((0, NPAD - N), (0, 0)))
    wl2p = jnp.pad(W_l2, ((0, GW - C), (0, 0)))
    wr2p = jnp.pad(W_r2, ((0, GW - C), (0, 0)))
    b2p = jnp.pad(b2, (0, GW - C)).reshape(1, GW)
    g, hr, ivb = _dense(
        embp, sum2, cnt2.reshape(NC * NPAD, 1),
        W_l1[:, :DH], W_l1[:, DH:], W_r1, b1.reshape(1, H), wl2p, wr2p, b2p)

    outq = _layer2(g, edges3, hr, ivb, nodesp)
    return (outq[0] + outq[1])[:B, :C]
